# Initial kernel scaffold; baseline (speedup 1.0000x reference)
import jax
import jax.numpy as jnp
from jax.experimental import pallas as pl


def kernel(batch_x, lap, l0_gate_w, l0_gate_b, l0_upd_w, l0_upd_b, l1_gate_w, l1_gate_b, l1_upd_w, l1_upd_b, node_emb, end_w, end_b):
    raise NotImplementedError("write your pallas kernel here")


# speedup vs baseline: 1.4241x; 1.4241x over previous
import jax
import jax.numpy as jnp
import numpy as np
from jax.experimental import pallas as pl
from jax.experimental.pallas import tpu as pltpu

B, T, N, H, D, K, OW = 8, 12, 128, 128, 12, 3, 12
NC = 2
BC = B // NC
BNC = BC * N
O2 = 2 * H
O3 = 3 * H


def _mm(a, w):
    return jnp.dot(a, w, preferred_element_type=jnp.float32)


def _fused_body(x_ref, sup_ref, e2_ref,
                wx0_ref, gwh0_ref, uwh0_ref, gb0_ref, ub0_ref,
                wx1_ref, gwh1_ref, uwh1_ref, gb1_ref, ub1_ref,
                ew_ref, eb_ref,
                out_ref, h1_scr, h2_scr):
    t = pl.program_id(1)
    nt = pl.num_programs(1)

    @pl.when(t == 0)
    def _init():
        h1_scr[...] = jnp.zeros_like(h1_scr)
        h2_scr[...] = jnp.zeros_like(h2_scr)

    sup = [jnp.broadcast_to(sup_ref[k][None], (BC, N, N)) for k in range(K)]
    e2 = e2_ref[...]

    def agg_cat(v2):
        f = v2.shape[-1]
        v3 = v2.reshape(BC, N, f)
        ys = [
            jnp.einsum("bnm,bmf->bnf", sup[k], v3,
                       preferred_element_type=jnp.float32).reshape(BNC, f)
            for k in range(K)
        ]
        return jnp.concatenate(ys, axis=-1)

    def fold(t2, o):
        acc = t2[:, :o] * e2[:, 0:1]
        for d in range(1, D):
            acc = acc + t2[:, d * o:(d + 1) * o] * e2[:, d:d + 1]
        return acc

    def gru(xc, state, gwh, uwh, gb, ub):
        ys = agg_cat(state)
        tg = xc[:, :D * O2] + _mm(ys, gwh)
        zr = jax.nn.sigmoid(fold(tg, O2) + gb)
        z = zr[:, :H]
        r = zr[:, H:]
        yzs = agg_cat(z * state)
        tc = xc[:, D * O2:] + _mm(yzs, uwh)
        hc = jnp.tanh(fold(tc, H) + ub)
        return r * state + (1.0 - r) * hc

    x2 = x_ref[0]
    xc0 = _mm(agg_cat(x2), wx0_ref[...])
    h1 = gru(xc0, h1_scr[...], gwh0_ref[...], uwh0_ref[...],
             gb0_ref[...], ub0_ref[...])
    h1_scr[...] = h1

    xc1 = _mm(agg_cat(h1), wx1_ref[...])
    h2 = gru(xc1, h2_scr[...], gwh1_ref[...], uwh1_ref[...],
             gb1_ref[...], ub1_ref[...])
    h2_scr[...] = h2

    @pl.when(t == nt - 1)
    def _head():
        out_ref[...] = _mm(h2, ew_ref[...]) + eb_ref[...]


def _prep_layer(gate_w, upd_w, c):
    def pool2d(pool, lo, hi, o):
        return jnp.transpose(pool[:, :, lo:hi, :], (1, 2, 0, 3)).reshape(
            K * (hi - lo), D * o)

    gx = pool2d(gate_w, 0, c, O2)
    gh = pool2d(gate_w, c, c + H, O2)
    ux = pool2d(upd_w, 0, c, H)
    uh = pool2d(upd_w, c, c + H, H)
    wx = jnp.concatenate([gx, ux], axis=1)
    return wx, gh, uh


def kernel(batch_x, lap, l0_gate_w, l0_gate_b, l0_upd_w, l0_upd_b,
           l1_gate_w, l1_gate_b, l1_upd_w, l1_upd_b,
           node_emb, end_w, end_b):
    eye = jnp.eye(N, dtype=jnp.float32)
    apt = jax.nn.softmax(jax.nn.relu(node_emb @ node_emb.T), axis=1)
    supports = jnp.stack([eye, lap, apt], axis=0)

    x_seq = jnp.transpose(batch_x, (1, 0, 2)).reshape(T, B * N, 1)

    e2 = jnp.tile(node_emb, (B, 1))
    wx0, gwh0, uwh0 = _prep_layer(l0_gate_w, l0_upd_w, 1)
    wx1, gwh1, uwh1 = _prep_layer(l1_gate_w, l1_upd_w, H)
    gb0 = jnp.tile(node_emb @ l0_gate_b, (B, 1))
    ub0 = jnp.tile(node_emb @ l0_upd_b, (B, 1))
    gb1 = jnp.tile(node_emb @ l1_gate_b, (B, 1))
    ub1 = jnp.tile(node_emb @ l1_upd_b, (B, 1))
    ew = jnp.transpose(end_w)
    eb = end_b.reshape(1, OW)

    row = lambda i, t: (i, 0)
    full2 = lambda i, t: (0, 0)
    full3 = lambda i, t: (0, 0, 0)

    out2 = pl.pallas_call(
        _fused_body,
        grid=(NC, T),
        in_specs=[
            pl.BlockSpec((1, BNC, 1), lambda i, t: (t, i, 0)),
            pl.BlockSpec(supports.shape, full3),
            pl.BlockSpec((BNC, D), row),
            pl.BlockSpec(wx0.shape, full2),
            pl.BlockSpec(gwh0.shape, full2),
            pl.BlockSpec(uwh0.shape, full2),
            pl.BlockSpec((BNC, O2), row),
            pl.BlockSpec((BNC, H), row),
            pl.BlockSpec(wx1.shape, full2),
            pl.BlockSpec(gwh1.shape, full2),
            pl.BlockSpec(uwh1.shape, full2),
            pl.BlockSpec((BNC, O2), row),
            pl.BlockSpec((BNC, H), row),
            pl.BlockSpec(ew.shape, full2),
            pl.BlockSpec(eb.shape, full2),
        ],
        out_specs=pl.BlockSpec((BNC, OW), row),
        out_shape=jax.ShapeDtypeStruct((B * N, OW), jnp.float32),
        scratch_shapes=[pltpu.VMEM((BNC, H), jnp.float32),
                        pltpu.VMEM((BNC, H), jnp.float32)],
        compiler_params=pltpu.CompilerParams(
            dimension_semantics=("parallel", "arbitrary")),
    )(x_seq, supports, e2, wx0, gwh0, uwh0, gb0, ub0,
      wx1, gwh1, uwh1, gb1, ub1, ew, eb)

    return out2.reshape(B, N, OW).transpose(0, 2, 1)


# speedup vs baseline: 1.5221x; 1.0688x over previous
import jax
import jax.numpy as jnp
import numpy as np
from jax.experimental import pallas as pl
from jax.experimental.pallas import tpu as pltpu

B, T, N, H, D, K, OW = 8, 12, 128, 128, 12, 3, 12
NC = 2
BC = B // NC
BNC = BC * N
O2 = 2 * H
O3 = 3 * H


def _mm(a, w):
    return jnp.dot(a, w, preferred_element_type=jnp.float32)


def _mmb(a, w):
    return jnp.dot(a.astype(jnp.bfloat16), w,
                   preferred_element_type=jnp.float32)


def _fused_body(x_ref, sup_ref, e2_ref,
                wx0_ref, gwh0_ref, uwh0_ref, gb0_ref, ub0_ref,
                wx1_ref, gwh1_ref, uwh1_ref, gb1_ref, ub1_ref,
                ew_ref, eb_ref,
                out_ref, h1_scr, h2_scr):
    t = pl.program_id(1)
    nt = pl.num_programs(1)

    @pl.when(t == 0)
    def _init():
        h1_scr[...] = jnp.zeros_like(h1_scr)
        h2_scr[...] = jnp.zeros_like(h2_scr)

    sup = [jnp.broadcast_to(sup_ref[k][None], (BC, N, N)) for k in range(K)]
    e2 = e2_ref[...]

    def agg_cat(v2):
        f = v2.shape[-1]
        v3 = v2.reshape(BC, N, f)
        ys = [
            jnp.einsum("bnm,bmf->bnf", sup[k], v3,
                       preferred_element_type=jnp.float32).reshape(BNC, f)
            for k in range(K)
        ]
        return jnp.concatenate(ys, axis=-1)

    def fold(t2, o):
        acc = t2[:, :o] * e2[:, 0:1]
        for d in range(1, D):
            acc = acc + t2[:, d * o:(d + 1) * o] * e2[:, d:d + 1]
        return acc

    def gru(xc, state, gwh, uwh, gb, ub):
        ys = agg_cat(state)
        tg = xc[:, :D * O2] + _mmb(ys, gwh)
        zr = jax.nn.sigmoid(fold(tg, O2) + gb)
        z = zr[:, :H]
        r = zr[:, H:]
        yzs = agg_cat(z * state)
        tc = xc[:, D * O2:] + _mmb(yzs, uwh)
        hc = jnp.tanh(fold(tc, H) + ub)
        return r * state + (1.0 - r) * hc

    x2 = x_ref[0]
    xc0 = _mmb(agg_cat(x2), wx0_ref[...])
    h1 = gru(xc0, h1_scr[...], gwh0_ref[...], uwh0_ref[...],
             gb0_ref[...], ub0_ref[...])
    h1_scr[...] = h1

    xc1 = _mmb(agg_cat(h1), wx1_ref[...])
    h2 = gru(xc1, h2_scr[...], gwh1_ref[...], uwh1_ref[...],
             gb1_ref[...], ub1_ref[...])
    h2_scr[...] = h2

    @pl.when(t == nt - 1)
    def _head():
        out_ref[...] = _mm(h2, ew_ref[...]) + eb_ref[...]


def _prep_layer(gate_w, upd_w, c):
    def pool2d(pool, lo, hi, o):
        return jnp.transpose(pool[:, :, lo:hi, :], (1, 2, 0, 3)).reshape(
            K * (hi - lo), D * o)

    gx = pool2d(gate_w, 0, c, O2)
    gh = pool2d(gate_w, c, c + H, O2)
    ux = pool2d(upd_w, 0, c, H)
    uh = pool2d(upd_w, c, c + H, H)
    wx = jnp.concatenate([gx, ux], axis=1)
    return (wx.astype(jnp.bfloat16), gh.astype(jnp.bfloat16),
            uh.astype(jnp.bfloat16))


def kernel(batch_x, lap, l0_gate_w, l0_gate_b, l0_upd_w, l0_upd_b,
           l1_gate_w, l1_gate_b, l1_upd_w, l1_upd_b,
           node_emb, end_w, end_b):
    eye = jnp.eye(N, dtype=jnp.float32)
    apt = jax.nn.softmax(jax.nn.relu(node_emb @ node_emb.T), axis=1)
    supports = jnp.stack([eye, lap, apt], axis=0)

    x_seq = jnp.transpose(batch_x, (1, 0, 2)).reshape(T, B * N, 1)

    e2 = jnp.tile(node_emb, (B, 1))
    wx0, gwh0, uwh0 = _prep_layer(l0_gate_w, l0_upd_w, 1)
    wx1, gwh1, uwh1 = _prep_layer(l1_gate_w, l1_upd_w, H)
    gb0 = jnp.tile(node_emb @ l0_gate_b, (B, 1))
    ub0 = jnp.tile(node_emb @ l0_upd_b, (B, 1))
    gb1 = jnp.tile(node_emb @ l1_gate_b, (B, 1))
    ub1 = jnp.tile(node_emb @ l1_upd_b, (B, 1))
    ew = jnp.transpose(end_w)
    eb = end_b.reshape(1, OW)

    row = lambda i, t: (i, 0)
    full2 = lambda i, t: (0, 0)
    full3 = lambda i, t: (0, 0, 0)

    out2 = pl.pallas_call(
        _fused_body,
        grid=(NC, T),
        in_specs=[
            pl.BlockSpec((1, BNC, 1), lambda i, t: (t, i, 0)),
            pl.BlockSpec(supports.shape, full3),
            pl.BlockSpec((BNC, D), row),
            pl.BlockSpec(wx0.shape, full2),
            pl.BlockSpec(gwh0.shape, full2),
            pl.BlockSpec(uwh0.shape, full2),
            pl.BlockSpec((BNC, O2), row),
            pl.BlockSpec((BNC, H), row),
            pl.BlockSpec(wx1.shape, full2),
            pl.BlockSpec(gwh1.shape, full2),
            pl.BlockSpec(uwh1.shape, full2),
            pl.BlockSpec((BNC, O2), row),
            pl.BlockSpec((BNC, H), row),
            pl.BlockSpec(ew.shape, full2),
            pl.BlockSpec(eb.shape, full2),
        ],
        out_specs=pl.BlockSpec((BNC, OW), row),
        out_shape=jax.ShapeDtypeStruct((B * N, OW), jnp.float32),
        scratch_shapes=[pltpu.VMEM((BNC, H), jnp.float32),
                        pltpu.VMEM((BNC, H), jnp.float32)],
        compiler_params=pltpu.CompilerParams(
            dimension_semantics=("parallel", "arbitrary")),
    )(x_seq, supports, e2, wx0, gwh0, uwh0, gb0, ub0,
      wx1, gwh1, uwh1, gb1, ub1, ew, eb)

    return out2.reshape(B, N, OW).transpose(0, 2, 1)


# speedup vs baseline: 1.9423x; 1.2761x over previous
import jax
import jax.numpy as jnp
from jax.experimental import pallas as pl
from jax.experimental.pallas import tpu as pltpu

B, T, N, H, D, K, OW = 8, 12, 128, 128, 12, 3, 12
BN = B * N
O2 = 2 * H
O3 = 3 * H


def _mm(a, w):
    return jnp.dot(a, w, preferred_element_type=jnp.float32)


def _mmb(a, w):
    return jnp.dot(a.astype(jnp.bfloat16), w,
                   preferred_element_type=jnp.float32)


def _fused_body(x_ref, sup_ref, e2_ref,
                fw0_ref, gwh0_ref, uwh0_ref, gb0_ref, ub0_ref,
                wg1_ref, wc1_ref, gb1_ref, ub1_ref,
                ew_ref, eb_ref,
                out_ref, h1_scr, h2_scr):
    t = pl.program_id(0)
    nt = pl.num_programs(0)

    @pl.when(t == 0)
    def _init():
        h1_scr[...] = jnp.zeros_like(h1_scr)
        h2_scr[...] = jnp.zeros_like(h2_scr)

    sup = [jnp.broadcast_to(sup_ref[k][None], (B, N, N)) for k in range(K)]
    e2 = e2_ref[...]

    def agg_cat(v2):
        f = v2.shape[-1]
        v3 = v2.reshape(B, N, f)
        ys = [
            jnp.einsum("bnm,bmf->bnf", sup[k], v3,
                       preferred_element_type=jnp.float32).reshape(BN, f)
            for k in range(K)
        ]
        return jnp.concatenate(ys, axis=-1)

    def fold(t2, o):
        acc = t2[:, :o] * e2[:, 0:1]
        for d in range(1, D):
            acc = acc + t2[:, d * o:(d + 1) * o] * e2[:, d:d + 1]
        return acc

    x2 = x_ref[0]
    yx0 = agg_cat(x2)
    xf = (yx0[:, 0:1] * fw0_ref[0] + yx0[:, 1:2] * fw0_ref[1]
          + yx0[:, 2:3] * fw0_ref[2])

    s1 = h1_scr[...]
    tg = _mmb(agg_cat(s1), gwh0_ref[...])
    zr = jax.nn.sigmoid(fold(tg, O2) + gb0_ref[...] + xf[:, :O2])
    z = zr[:, :H]
    r = zr[:, H:]
    tc = _mmb(agg_cat(z * s1), uwh0_ref[...])
    hc = jnp.tanh(fold(tc, H) + ub0_ref[...] + xf[:, O2:])
    h1 = r * s1 + (1.0 - r) * hc
    h1_scr[...] = h1

    yx1 = agg_cat(h1)
    s2 = h2_scr[...]
    tg1 = _mmb(jnp.concatenate([yx1, agg_cat(s2)], axis=-1), wg1_ref[...])
    zr1 = jax.nn.sigmoid(fold(tg1, O2) + gb1_ref[...])
    z1 = zr1[:, :H]
    r1 = zr1[:, H:]
    tc1 = _mmb(jnp.concatenate([yx1, agg_cat(z1 * s2)], axis=-1), wc1_ref[...])
    hc1 = jnp.tanh(fold(tc1, H) + ub1_ref[...])
    h2 = r1 * s2 + (1.0 - r1) * hc1
    h2_scr[...] = h2

    @pl.when(t == nt - 1)
    def _head():
        out_ref[...] = _mm(h2, ew_ref[...]) + eb_ref[...]


def _pool2d(pool, lo, hi, o):
    return jnp.transpose(pool[:, :, lo:hi, :], (1, 2, 0, 3)).reshape(
        K * (hi - lo), D * o)


def kernel(batch_x, lap, l0_gate_w, l0_gate_b, l0_upd_w, l0_upd_b,
           l1_gate_w, l1_gate_b, l1_upd_w, l1_upd_b,
           node_emb, end_w, end_b):
    eye = jnp.eye(N, dtype=jnp.float32)
    apt = jax.nn.softmax(jax.nn.relu(node_emb @ node_emb.T), axis=1)
    supports = jnp.stack([eye, lap, apt], axis=0)

    x_seq = jnp.transpose(batch_x, (1, 0, 2)).reshape(T, BN, 1)
    e2 = jnp.tile(node_emb, (B, 1))

    gx0 = _pool2d(l0_gate_w, 0, 1, O2).reshape(K, D, O2)
    ux0 = _pool2d(l0_upd_w, 0, 1, H).reshape(K, D, H)
    fwg = jnp.einsum("nd,kdo->kno", node_emb, gx0)
    fwu = jnp.einsum("nd,kdo->kno", node_emb, ux0)
    fw0 = jnp.tile(jnp.concatenate([fwg, fwu], axis=-1), (1, B, 1))
    gwh0 = _pool2d(l0_gate_w, 1, 1 + H, O2).astype(jnp.bfloat16)
    uwh0 = _pool2d(l0_upd_w, 1, 1 + H, H).astype(jnp.bfloat16)

    wg1 = jnp.concatenate([_pool2d(l1_gate_w, 0, H, O2),
                           _pool2d(l1_gate_w, H, 2 * H, O2)],
                          axis=0).astype(jnp.bfloat16)
    wc1 = jnp.concatenate([_pool2d(l1_upd_w, 0, H, H),
                           _pool2d(l1_upd_w, H, 2 * H, H)],
                          axis=0).astype(jnp.bfloat16)

    gb0 = jnp.tile(node_emb @ l0_gate_b, (B, 1))
    ub0 = jnp.tile(node_emb @ l0_upd_b, (B, 1))
    gb1 = jnp.tile(node_emb @ l1_gate_b, (B, 1))
    ub1 = jnp.tile(node_emb @ l1_upd_b, (B, 1))
    ew = jnp.transpose(end_w)
    eb = end_b.reshape(1, OW)

    full2 = lambda t: (0, 0)
    full3 = lambda t: (0, 0, 0)

    out2 = pl.pallas_call(
        _fused_body,
        grid=(T,),
        in_specs=[
            pl.BlockSpec((1, BN, 1), lambda t: (t, 0, 0)),
            pl.BlockSpec(supports.shape, full3),
            pl.BlockSpec(e2.shape, full2),
            pl.BlockSpec(fw0.shape, full3),
            pl.BlockSpec(gwh0.shape, full2),
            pl.BlockSpec(uwh0.shape, full2),
            pl.BlockSpec(gb0.shape, full2),
            pl.BlockSpec(ub0.shape, full2),
            pl.BlockSpec(wg1.shape, full2),
            pl.BlockSpec(wc1.shape, full2),
            pl.BlockSpec(gb1.shape, full2),
            pl.BlockSpec(ub1.shape, full2),
            pl.BlockSpec(ew.shape, full2),
            pl.BlockSpec(eb.shape, full2),
        ],
        out_specs=pl.BlockSpec((BN, OW), full2),
        out_shape=jax.ShapeDtypeStruct((BN, OW), jnp.float32),
        scratch_shapes=[pltpu.VMEM((BN, H), jnp.float32),
                        pltpu.VMEM((BN, H), jnp.float32)],
        compiler_params=pltpu.CompilerParams(
            dimension_semantics=("arbitrary",)),
    )(x_seq, supports, e2, fw0, gwh0, uwh0, gb0, ub0,
      wg1, wc1, gb1, ub1, ew, eb)

    return out2.reshape(B, N, OW).transpose(0, 2, 1)


# speedup vs baseline: 2.1148x; 1.0888x over previous
import jax
import jax.numpy as jnp
from jax.experimental import pallas as pl
from jax.experimental.pallas import tpu as pltpu

B, T, N, H, D, K, OW = 8, 12, 128, 128, 12, 3, 12
BN = B * N
O2 = 2 * H
O3 = 3 * H


def _mmb(a, w):
    return jnp.dot(a.reshape(BN, a.shape[-1]).astype(jnp.bfloat16), w,
                   preferred_element_type=jnp.float32)


def _fused_body(x_ref, sup_ref, emb_ref, fw0_ref,
                gwh0_ref, uwh0_ref, wg1_ref, wc1_ref, bias_ref,
                ew_ref, eb_ref, out_ref, h1_scr, h2_scr):
    t = pl.program_id(0)
    nt = pl.num_programs(0)

    @pl.when(t == 0)
    def _init():
        h1_scr[...] = jnp.zeros_like(h1_scr)
        h2_scr[...] = jnp.zeros_like(h2_scr)

    sup = [jnp.broadcast_to(sup_ref[k][None], (B, N, N)) for k in range(K)]
    emb = emb_ref[...]
    bias = bias_ref[...][None]

    def agg_cat(v3):
        ys = [
            jnp.einsum("bnm,bmf->bnf", sup[k], v3,
                       preferred_element_type=jnp.float32)
            for k in range(K)
        ]
        return jnp.concatenate(ys, axis=-1)

    def fold(t2, o):
        t3 = t2.reshape(B, N, D * o)
        acc = t3[..., :o] * emb[None, :, 0:1]
        for d in range(1, D):
            acc = acc + t3[..., d * o:(d + 1) * o] * emb[None, :, d:d + 1]
        return acc

    x3 = x_ref[...][..., None]
    yx0 = agg_cat(x3)
    xf = (yx0[..., 0:1] * fw0_ref[0][None]
          + yx0[..., 1:2] * fw0_ref[1][None]
          + yx0[..., 2:3] * fw0_ref[2][None])

    s1 = h1_scr[...].reshape(B, N, H)
    tg = _mmb(agg_cat(s1), gwh0_ref[...])
    zr = jax.nn.sigmoid(fold(tg, O2) + bias[..., :O2] + xf[..., :O2])
    z = zr[..., :H]
    r = zr[..., H:]
    tc = _mmb(agg_cat(z * s1), uwh0_ref[...])
    hc = jnp.tanh(fold(tc, H) + bias[..., O2:O3] + xf[..., O2:])
    h1 = r * s1 + (1.0 - r) * hc
    h1_scr[...] = h1.reshape(BN, H)

    yx1 = agg_cat(h1)
    s2 = h2_scr[...].reshape(B, N, H)
    tg1 = _mmb(jnp.concatenate([yx1, agg_cat(s2)], axis=-1), wg1_ref[...])
    zr1 = jax.nn.sigmoid(fold(tg1, O2) + bias[..., O3:O3 + O2])
    z1 = zr1[..., :H]
    r1 = zr1[..., H:]
    tc1 = _mmb(jnp.concatenate([yx1, agg_cat(z1 * s2)], axis=-1), wc1_ref[...])
    hc1 = jnp.tanh(fold(tc1, H) + bias[..., O3 + O2:])
    h2 = r1 * s2 + (1.0 - r1) * hc1
    h2_scr[...] = h2.reshape(BN, H)

    @pl.when(t == nt - 1)
    def _head():
        o3 = jnp.dot(h2.reshape(BN, H), ew_ref[...],
                     preferred_element_type=jnp.float32) + eb_ref[...]
        out_ref[...] = jnp.transpose(
            o3.reshape(B, N, OW), (0, 2, 1))


def _pool2d(pool, lo, hi, o):
    return jnp.transpose(pool[:, :, lo:hi, :], (1, 2, 0, 3)).reshape(
        K * (hi - lo), D * o)


def kernel(batch_x, lap, l0_gate_w, l0_gate_b, l0_upd_w, l0_upd_b,
           l1_gate_w, l1_gate_b, l1_upd_w, l1_upd_b,
           node_emb, end_w, end_b):
    eye = jnp.eye(N, dtype=jnp.float32)
    apt = jax.nn.softmax(jax.nn.relu(node_emb @ node_emb.T), axis=1)
    supports = jnp.stack([eye, lap, apt], axis=0)

    wx0 = jnp.concatenate(
        [l0_gate_w[:, :, 0, :], l0_upd_w[:, :, 0, :]], axis=-1)
    fw0 = jnp.einsum("nd,dko->kno", node_emb, wx0)
    gwh0 = _pool2d(l0_gate_w, 1, 1 + H, O2).astype(jnp.bfloat16)
    uwh0 = _pool2d(l0_upd_w, 1, 1 + H, H).astype(jnp.bfloat16)

    wg1 = jnp.concatenate([_pool2d(l1_gate_w, 0, H, O2),
                           _pool2d(l1_gate_w, H, 2 * H, O2)],
                          axis=0).astype(jnp.bfloat16)
    wc1 = jnp.concatenate([_pool2d(l1_upd_w, 0, H, H),
                           _pool2d(l1_upd_w, H, 2 * H, H)],
                          axis=0).astype(jnp.bfloat16)

    bias = node_emb @ jnp.concatenate(
        [l0_gate_b, l0_upd_b, l1_gate_b, l1_upd_b], axis=1)

    ew = jnp.transpose(end_w)
    eb = end_b.reshape(1, OW)

    full2 = lambda t: (0, 0)
    full3 = lambda t: (0, 0, 0)

    out = pl.pallas_call(
        _fused_body,
        grid=(T,),
        in_specs=[
            pl.BlockSpec((B, N), lambda t: (0, t)),
            pl.BlockSpec(supports.shape, full3),
            pl.BlockSpec(node_emb.shape, full2),
            pl.BlockSpec(fw0.shape, full3),
            pl.BlockSpec(gwh0.shape, full2),
            pl.BlockSpec(uwh0.shape, full2),
            pl.BlockSpec(wg1.shape, full2),
            pl.BlockSpec(wc1.shape, full2),
            pl.BlockSpec(bias.shape, full2),
            pl.BlockSpec(ew.shape, full2),
            pl.BlockSpec(eb.shape, full2),
        ],
        out_specs=pl.BlockSpec((B, OW, N), full3),
        out_shape=jax.ShapeDtypeStruct((B, OW, N), jnp.float32),
        scratch_shapes=[pltpu.VMEM((BN, H), jnp.float32),
                        pltpu.VMEM((BN, H), jnp.float32)],
        compiler_params=pltpu.CompilerParams(
            dimension_semantics=("arbitrary",)),
    )(batch_x.reshape(B, T * N), supports, node_emb, fw0,
      gwh0, uwh0, wg1, wc1, bias, ew, eb)

    return out
